# baseline (speedup 1.0000x reference)
import functools

import jax
import jax.numpy as jnp
from jax.experimental import pallas as pl
from jax.experimental.pallas import tpu as pltpu

_BM = 400
_NBUF = 3


def _gcn_kernel(adj_hbm, x_hbm, w_ref, b_ref, out_ref,
                bufs, xbuf, h_ref, sems, xsem, *, n, nblk):
    def adj_copy(i, slot):
        return pltpu.make_async_copy(
            adj_hbm.at[pl.ds(i * _BM, _BM), :], bufs.at[slot], sems.at[slot])

    for s in range(_NBUF):
        adj_copy(s, s).start()
    xcp = pltpu.make_async_copy(x_hbm, xbuf, xsem)
    xcp.start()
    xcp.wait()
    h = jax.lax.dot_general(
        xbuf[...], w_ref[...],
        dimension_numbers=(((1,), (1,)), ((), ())),
        preferred_element_type=jnp.float32,
    )
    h_ref[...] = h.astype(jnp.bfloat16)

    def step(i, slot):
        adj_copy(i, slot).wait()
        acc = jax.lax.dot_general(
            bufs[slot].astype(jnp.bfloat16), h_ref[...],
            dimension_numbers=(((1,), (0,)), ((), ())),
            preferred_element_type=jnp.float32,
        )
        out_ref[pl.ds(i * _BM, _BM), :] = acc + b_ref[...]

    def outer(o, carry):
        for s in range(_NBUF):
            i = o * _NBUF + s
            step(i, s)

            @pl.when(i + _NBUF < nblk)
            def _():
                adj_copy(i + _NBUF, s).start()
        return carry

    jax.lax.fori_loop(0, nblk // _NBUF, outer, 0, unroll=False)
    for i in range((nblk // _NBUF) * _NBUF, nblk):
        step(i, i % _NBUF)


def kernel(x, adj, W, b):
    n, in_ch = x.shape
    out_ch = W.shape[0]
    nblk = n // _BM

    b2 = b.reshape(1, out_ch)
    out = pl.pallas_call(
        functools.partial(_gcn_kernel, n=n, nblk=nblk),
        in_specs=[
            pl.BlockSpec(memory_space=pl.ANY),
            pl.BlockSpec(memory_space=pl.ANY),
            pl.BlockSpec(memory_space=pltpu.VMEM),
            pl.BlockSpec(memory_space=pltpu.VMEM),
        ],
        out_specs=pl.BlockSpec(memory_space=pltpu.VMEM),
        out_shape=jax.ShapeDtypeStruct((n, out_ch), jnp.float32),
        scratch_shapes=[
            pltpu.VMEM((_NBUF, _BM, n), jnp.float32),
            pltpu.VMEM((n, in_ch), jnp.float32),
            pltpu.VMEM((n, out_ch), jnp.bfloat16),
            pltpu.SemaphoreType.DMA((_NBUF,)),
            pltpu.SemaphoreType.DMA,
        ],
        compiler_params=pltpu.CompilerParams(
            vmem_limit_bytes=64 * 1024 * 1024,
        ),
    )(adj, x, W, b2)
    return out


# speedup vs baseline: 1.0760x; 1.0760x over previous
import jax
import jax.numpy as jnp
from jax.experimental import pallas as pl
from jax.experimental.pallas import tpu as pltpu


def _gcn_kernel(adj_ref, x_ref, w_ref, b_ref, out_ref, h_ref):
    @pl.when(pl.program_id(0) == 0)
    def _():
        h = jax.lax.dot_general(
            x_ref[...], w_ref[...],
            dimension_numbers=(((1,), (1,)), ((), ())),
            preferred_element_type=jnp.float32,
        )
        h_ref[...] = h.astype(jnp.bfloat16)

    acc = jax.lax.dot_general(
        adj_ref[...].astype(jnp.bfloat16), h_ref[...],
        dimension_numbers=(((1,), (0,)), ((), ())),
        preferred_element_type=jnp.float32,
    )
    out_ref[...] = acc + b_ref[...]


def kernel(x, adj, W, b):
    n, in_ch = x.shape
    out_ch = W.shape[0]
    bm = 400

    b2 = b.reshape(1, out_ch)
    out = pl.pallas_call(
        _gcn_kernel,
        grid=(n // bm,),
        in_specs=[
            pl.BlockSpec((bm, n), lambda i: (i, 0)),
            pl.BlockSpec((n, in_ch), lambda i: (0, 0)),
            pl.BlockSpec((out_ch, in_ch), lambda i: (0, 0)),
            pl.BlockSpec((1, out_ch), lambda i: (0, 0)),
        ],
        out_specs=pl.BlockSpec((bm, out_ch), lambda i: (i, 0)),
        out_shape=jax.ShapeDtypeStruct((n, out_ch), jnp.float32),
        scratch_shapes=[pltpu.VMEM((n, out_ch), jnp.bfloat16)],
        compiler_params=pltpu.CompilerParams(
            dimension_semantics=("arbitrary",),
        ),
    )(adj, x, W, b2)
    return out
